# Initial kernel scaffold; baseline (speedup 1.0000x reference)
import jax
import jax.numpy as jnp
from jax.experimental import pallas as pl


def kernel(x, weight):
    raise NotImplementedError("write your pallas kernel here")


# speedup vs baseline: 1.1121x; 1.1121x over previous
import functools

import jax
import jax.numpy as jnp
from jax import lax
from jax.experimental import pallas as pl
from jax.experimental.pallas import tpu as pltpu, tpu_sc as plsc


CHUNK = 128
K = 10


@functools.cache
def _make_gather(n, v, d):
    info = plsc.get_sparse_core_info()
    nc, ns = info.num_cores, info.num_subcores
    nw = nc * ns
    assert n % (nw * CHUNK) == 0
    npw = n // nw
    cpw = npw // CHUNK
    assert cpw % (2 * K) == 0
    groups = cpw // K
    npair = groups // 2
    rpg = K * CHUNK

    mesh = plsc.VectorSubcoreMesh(core_axis_name="c", subcore_axis_name="s")

    @functools.partial(
        pl.kernel,
        mesh=mesh,
        compiler_params=pltpu.CompilerParams(use_tc_tiling_on_sc=False),
        out_type=jax.ShapeDtypeStruct((n, d), jnp.float32),
        scratch_types=[
            pltpu.VMEM((cpw, CHUNK), jnp.int32),
            pltpu.VMEM((rpg, d), jnp.float32),
            pltpu.VMEM((rpg, d), jnp.float32),
            pltpu.SemaphoreType.DMA,
            pltpu.SemaphoreType.DMA,
        ],
    )
    def gather(table_hbm, idx_hbm, out_hbm, idx_v, rows0, rows1, sem0, sem1):
        wid = lax.axis_index("s") * nc + lax.axis_index("c")
        base = wid * npw

        pltpu.sync_copy(idx_hbm.at[pl.ds(wid * cpw, cpw)], idx_v)

        def fire(g, rows, sem):
            c0 = g * K
            for b in range(K):
                pltpu.async_copy(
                    table_hbm.at[idx_v.at[c0 + b]],
                    rows.at[pl.ds(b * CHUNK, CHUNK)],
                    sem,
                )

        def drain(rows, sem):
            pltpu.make_async_copy(table_hbm.at[pl.ds(0, rpg)], rows, sem).wait()

        def flush(g, rows):
            pltpu.sync_copy(rows, out_hbm.at[pl.ds(base + g * rpg, rpg)])

        fire(0, rows0, sem0)

        def pair(p, carry):
            g0 = 2 * p
            fire(g0 + 1, rows1, sem1)
            drain(rows0, sem0)
            flush(g0, rows0)

            @pl.when(p < npair - 1)
            def _():
                fire(g0 + 2, rows0, sem0)

            drain(rows1, sem1)
            flush(g0 + 1, rows1)
            return carry

        lax.fori_loop(0, npair, pair, 0)

    return gather


def kernel(x, weight):
    b, h = x.shape
    v, d = weight.shape
    n = b * h
    idx = x.reshape(n // CHUNK, CHUNK).astype(jnp.int32)
    out = _make_gather(n, v, d)(weight, idx)
    return out.reshape(b, h, d)


# speedup vs baseline: 1.6047x; 1.4430x over previous
import functools

import jax
import jax.numpy as jnp
from jax import lax
from jax.experimental import pallas as pl
from jax.experimental.pallas import tpu as pltpu, tpu_sc as plsc


@functools.cache
def _make_gather(batch, hist, v, d):
    info = plsc.get_sparse_core_info()
    nc, ns = info.num_cores, info.num_subcores
    nw = nc * ns
    bc = batch // nw
    ndesc = bc // 128
    dt = d // 8
    assert bc % 128 == 0 and d % 8 == 0 and batch % 128 == 0 and hist % 2 == 0

    mesh = plsc.VectorSubcoreMesh(core_axis_name="c", subcore_axis_name="s")

    @functools.partial(
        pl.kernel,
        mesh=mesh,
        compiler_params=pltpu.CompilerParams(
            use_tc_tiling_on_sc=False, needs_layout_passes=False),
        out_type=jax.ShapeDtypeStruct((hist, dt, batch // 128, 8, 128),
                                      jnp.float32),
        scratch_types=[
            pltpu.VMEM((hist, ndesc, 128), jnp.int32),
            pltpu.VMEM((bc, d), jnp.float32),
            pltpu.VMEM((bc, d), jnp.float32),
            pltpu.VMEM((dt, ndesc, 8, 128), jnp.float32),
            pltpu.SemaphoreType.DMA,
            pltpu.SemaphoreType.DMA,
        ],
    )
    def gather(table_hbm, x3_hbm, out_hbm, idx_v, rows0, rows1, tr_v,
               sem0, sem1):
        wid = lax.axis_index("s") * nc + lax.axis_index("c")

        pltpu.sync_copy(x3_hbm.at[:, pl.ds(wid * ndesc, ndesc), :], idx_v)

        viota = lax.iota(jnp.int32, 16)
        cols = [jnp.full((16,), c, jnp.int32) for c in range(d)]

        def fire(h, rows, sem):
            for j in range(ndesc):
                pltpu.async_copy(
                    table_hbm.at[idx_v.at[h, j]],
                    rows.at[pl.ds(j * 128, 128)],
                    sem,
                )

        def drain(rows, sem):
            pltpu.make_async_copy(table_hbm.at[pl.ds(0, bc)], rows, sem).wait()

        def trans_flush(h, rows):
            def blk_body(blk, carry):
                ridx = viota + blk * 16
                ct = blk // 8
                off = (blk % 8) * 16
                for dd in range(d):
                    vals = plsc.load_gather(rows, [ridx, cols[dd]])
                    tr_v[dd // 8, ct, dd % 8, pl.ds(off, 16)] = vals
                return carry

            lax.fori_loop(0, bc // 16, blk_body, 0)
            pltpu.sync_copy(
                tr_v, out_hbm.at[h, :, pl.ds(wid * ndesc, ndesc), :, :])

        fire(0, rows0, sem0)

        def pair(p, carry):
            h0 = 2 * p
            fire(h0 + 1, rows1, sem1)
            drain(rows0, sem0)
            trans_flush(h0, rows0)

            @pl.when(p < hist // 2 - 1)
            def _():
                fire(h0 + 2, rows0, sem0)

            drain(rows1, sem1)
            trans_flush(h0 + 1, rows1)
            return carry

        lax.fori_loop(0, hist // 2, pair, 0)

    return gather


def kernel(x, weight):
    b, h = x.shape
    v, d = weight.shape
    x3 = x.T.reshape(h, b // 128, 128).astype(jnp.int32)
    out5 = _make_gather(b, h, v, d)(weight, x3)
    return out5.transpose(2, 4, 0, 1, 3).reshape(b, h, d)


# speedup vs baseline: 2.1256x; 1.3246x over previous
import functools

import jax
import jax.numpy as jnp
from jax import lax
from jax.experimental import pallas as pl
from jax.experimental.pallas import tpu as pltpu, tpu_sc as plsc


@functools.cache
def _make_gather(batch, hist, v, d):
    info = plsc.get_sparse_core_info()
    nc, ns = info.num_cores, info.num_subcores
    nw = nc * ns
    bc = batch // nw
    ndesc = bc // 128
    dt = d // 8
    assert bc % 128 == 0 and d % 8 == 0 and batch % 128 == 0 and hist % 2 == 0

    mesh = plsc.VectorSubcoreMesh(core_axis_name="c", subcore_axis_name="s")

    @functools.partial(
        pl.kernel,
        mesh=mesh,
        compiler_params=pltpu.CompilerParams(
            use_tc_tiling_on_sc=False, needs_layout_passes=False),
        out_type=jax.ShapeDtypeStruct((hist, dt, batch // 128, 8, 128),
                                      jnp.float32),
        scratch_types=[
            pltpu.VMEM((hist, ndesc, 128), jnp.int32),
            pltpu.VMEM((bc, d), jnp.float32),
            pltpu.VMEM((bc, d), jnp.float32),
            pltpu.VMEM((dt, ndesc, 8, 128), jnp.float32),
            pltpu.VMEM((dt, ndesc, 8, 128), jnp.float32),
            pltpu.SemaphoreType.DMA,
            pltpu.SemaphoreType.DMA,
            pltpu.SemaphoreType.DMA,
            pltpu.SemaphoreType.DMA,
        ],
    )
    def gather(table_hbm, x3_hbm, out_hbm, idx_v, rows0, rows1, tr0, tr1,
               sem0, sem1, semf0, semf1):
        wid = lax.axis_index("s") * nc + lax.axis_index("c")

        pltpu.sync_copy(x3_hbm.at[:, pl.ds(wid * ndesc, ndesc), :], idx_v)

        viota = lax.iota(jnp.int32, 16)

        def fire(h, rows, sem):
            for j in range(ndesc):
                pltpu.async_copy(
                    table_hbm.at[idx_v.at[h, j]],
                    rows.at[pl.ds(j * 128, 128)],
                    sem,
                )

        def drain(rows, sem):
            pltpu.make_async_copy(table_hbm.at[pl.ds(0, bc)], rows, sem).wait()

        def out_slab(h):
            return out_hbm.at[h, :, pl.ds(wid * ndesc, ndesc), :, :]

        def trans(rows, tr):
            def blk_body(blk, carry):
                ridx = viota + blk * 16
                ct = blk // 8
                off = (blk % 8) * 16
                vals = plsc.load_gather(
                    rows, [ridx, jnp.full((16,), 0, jnp.int32)])
                for dd in range(1, d):
                    nxt = plsc.load_gather(
                        rows, [ridx, jnp.full((16,), dd, jnp.int32)])
                    tr[(dd - 1) // 8, ct, (dd - 1) % 8, pl.ds(off, 16)] = vals
                    vals = nxt
                tr[(d - 1) // 8, ct, (d - 1) % 8, pl.ds(off, 16)] = vals
                return carry

            lax.fori_loop(0, bc // 16, blk_body, 0)

        fire(0, rows0, sem0)

        def pair(p, carry):
            h0 = 2 * p
            fire(h0 + 1, rows1, sem1)
            drain(rows0, sem0)

            @pl.when(p > 0)
            def _():
                pltpu.make_async_copy(tr0, out_slab(h0), semf0).wait()

            trans(rows0, tr0)
            pltpu.async_copy(tr0, out_slab(h0), semf0)

            @pl.when(p < hist // 2 - 1)
            def _():
                fire(h0 + 2, rows0, sem0)

            drain(rows1, sem1)

            @pl.when(p > 0)
            def _():
                pltpu.make_async_copy(tr1, out_slab(h0 + 1), semf1).wait()

            trans(rows1, tr1)
            pltpu.async_copy(tr1, out_slab(h0 + 1), semf1)
            return carry

        lax.fori_loop(0, hist // 2, pair, 0)
        pltpu.make_async_copy(tr0, out_slab(hist - 2), semf0).wait()
        pltpu.make_async_copy(tr1, out_slab(hist - 1), semf1).wait()

    return gather


def kernel(x, weight):
    b, h = x.shape
    v, d = weight.shape
    x3 = x.T.reshape(h, b // 128, 128).astype(jnp.int32)
    out5 = _make_gather(b, h, v, d)(weight, x3)
    return out5.transpose(2, 4, 0, 1, 3).reshape(b, h, d)


# speedup vs baseline: 2.1558x; 1.0142x over previous
import functools

import jax
import jax.numpy as jnp
from jax import lax
from jax.experimental import pallas as pl
from jax.experimental.pallas import tpu as pltpu, tpu_sc as plsc


@functools.cache
def _make_gather(batch, hist, v, d):
    info = plsc.get_sparse_core_info()
    nc, ns = info.num_cores, info.num_subcores
    nw = nc * ns
    bc = batch // nw
    ndesc = bc // 128
    dt = d // 8
    assert bc % 128 == 0 and d % 8 == 0 and batch % 128 == 0 and hist % 2 == 0

    mesh = plsc.VectorSubcoreMesh(core_axis_name="c", subcore_axis_name="s")

    @functools.partial(
        pl.kernel,
        mesh=mesh,
        compiler_params=pltpu.CompilerParams(
            use_tc_tiling_on_sc=False, needs_layout_passes=False),
        out_type=jax.ShapeDtypeStruct((hist, dt, batch // 128, 8, 128),
                                      jnp.float32),
        scratch_types=[
            pltpu.VMEM((hist, ndesc, 128), jnp.int32),
            pltpu.VMEM((bc, d), jnp.float32),
            pltpu.VMEM((bc, d), jnp.float32),
            pltpu.VMEM((dt, ndesc, 8, 128), jnp.float32),
            pltpu.VMEM((dt, ndesc, 8, 128), jnp.float32),
            pltpu.SemaphoreType.DMA,
            pltpu.SemaphoreType.DMA,
            pltpu.SemaphoreType.DMA,
            pltpu.SemaphoreType.DMA,
        ],
    )
    def gather(table_hbm, x3_hbm, out_hbm, idx_v, rows0, rows1, tr0, tr1,
               sem0, sem1, semf0, semf1):
        wid = lax.axis_index("s") * nc + lax.axis_index("c")

        pltpu.sync_copy(x3_hbm.at[:, pl.ds(wid * ndesc, ndesc), :], idx_v)

        viota = lax.iota(jnp.int32, 16)

        def fire(h, rows, sem):
            for j in range(ndesc):
                pltpu.async_copy(
                    table_hbm.at[idx_v.at[h, j]],
                    rows.at[pl.ds(j * 128, 128)],
                    sem,
                )

        def drain(rows, sem):
            pltpu.make_async_copy(table_hbm.at[pl.ds(0, bc)], rows, sem).wait()

        def out_slab(h):
            return out_hbm.at[h, :, pl.ds(wid * ndesc, ndesc), :, :]

        def trans(rows, tr):
            def blk_body(blk, carry):
                ridx = viota + blk * 16
                ct = blk // 8
                off = (blk % 8) * 16
                v0 = plsc.load_gather(
                    rows, [ridx, jnp.full((16,), 0, jnp.int32)])
                v1 = plsc.load_gather(
                    rows, [ridx, jnp.full((16,), 1, jnp.int32)])
                for dd in range(2, d):
                    nxt = plsc.load_gather(
                        rows, [ridx, jnp.full((16,), dd, jnp.int32)])
                    tr[(dd - 2) // 8, ct, (dd - 2) % 8, pl.ds(off, 16)] = v0
                    v0, v1 = v1, nxt
                tr[(d - 2) // 8, ct, (d - 2) % 8, pl.ds(off, 16)] = v0
                tr[(d - 1) // 8, ct, (d - 1) % 8, pl.ds(off, 16)] = v1
                return carry

            lax.fori_loop(0, bc // 16, blk_body, 0)

        fire(0, rows0, sem0)

        def pair(p, carry):
            h0 = 2 * p
            fire(h0 + 1, rows1, sem1)
            drain(rows0, sem0)

            @pl.when(p > 0)
            def _():
                pltpu.make_async_copy(tr0, out_slab(h0), semf0).wait()

            trans(rows0, tr0)
            pltpu.async_copy(tr0, out_slab(h0), semf0)

            @pl.when(p < hist // 2 - 1)
            def _():
                fire(h0 + 2, rows0, sem0)

            drain(rows1, sem1)

            @pl.when(p > 0)
            def _():
                pltpu.make_async_copy(tr1, out_slab(h0 + 1), semf1).wait()

            trans(rows1, tr1)
            pltpu.async_copy(tr1, out_slab(h0 + 1), semf1)
            return carry

        lax.fori_loop(0, hist // 2, pair, 0)
        pltpu.make_async_copy(tr0, out_slab(hist - 2), semf0).wait()
        pltpu.make_async_copy(tr1, out_slab(hist - 1), semf1).wait()

    return gather


def kernel(x, weight):
    b, h = x.shape
    v, d = weight.shape
    x3 = x.T.reshape(h, b // 128, 128).astype(jnp.int32)
    out5 = _make_gather(b, h, v, d)(weight, x3)
    return out5.transpose(2, 4, 0, 1, 3).reshape(b, h, d)
